# baseline (speedup 1.0000x reference)
import functools

import jax
import jax.numpy as jnp
from jax import lax
from jax.experimental import pallas as pl
from jax.experimental.pallas import tpu as pltpu
from jax.experimental.pallas import tpu_sc as plsc

LANES = 16
GROUP = 8
TILE_W = 128


def _make_gather(b: int, k: int, n: int):
    info = plsc.get_sparse_core_info()
    nc, ns = info.num_cores, info.num_subcores
    nw = nc * ns
    b_per_w = b // nw
    n_groups = b_per_w // GROUP
    mesh = plsc.VectorSubcoreMesh(core_axis_name="c", subcore_axis_name="s")

    @functools.partial(
        pl.kernel,
        mesh=mesh,
        out_type=jax.ShapeDtypeStruct((k, b), jnp.float32),
        scratch_types=[
            pltpu.VMEM((b_per_w + LANES,), jnp.int32),
            pltpu.VMEM((2, GROUP, k, TILE_W), jnp.float32),
            pltpu.VMEM((k, b_per_w), jnp.float32),
            pltpu.SemaphoreType.DMA,
            pltpu.SemaphoreType.DMA,
        ],
        compiler_params=pltpu.CompilerParams(
            disable_bounds_checks=True, needs_layout_passes=False
        ),
    )
    def gather_kernel(
        table_hbm, idx_hbm, out_hbm, idx_v, blk_v, cols_v, sem0, sem1
    ):
        wid = lax.axis_index("s") * nc + lax.axis_index("c")
        base = wid * b_per_w
        pltpu.sync_copy(
            idx_hbm.at[pl.ds(base, b_per_w)], idx_v.at[pl.ds(0, b_per_w)]
        )
        lane_iota = lax.iota(jnp.int32, LANES)
        lo_mask = lane_iota < GROUP
        blk_lane = lane_iota & (GROUP - 1)
        sems = (sem0, sem1)

        def fire(g, buf, sem):
            vec = idx_v[pl.ds(g * GROUP, LANES)]
            copies = []
            for l in range(GROUP):
                idx_l = lax.squeeze(lax.slice(vec, (l,), (l + 1,)), (0,))
                toff = pl.multiple_of((idx_l // TILE_W) * TILE_W, TILE_W)
                copies.append(
                    pltpu.async_copy(
                        table_hbm.at[:, pl.ds(toff, TILE_W)],
                        blk_v.at[buf, l],
                        sem,
                    )
                )
            return copies

        def drain_extract(g, buf, copies):
            for c in copies:
                c.wait()
            vec = idx_v[pl.ds(g * GROUP, LANES)]
            cvec = lax.rem(vec, TILE_W)
            obase = g * GROUP
            for j in range(k):
                jvec = jnp.full((LANES,), j, jnp.int32)
                vals = plsc.load_gather(
                    blk_v.at[buf], [blk_lane, jvec, cvec], mask=lo_mask
                )
                plsc.store_scatter(
                    cols_v,
                    [jvec, blk_lane + obase],
                    vals,
                    mask=lo_mask,
                )

        def body(m, carry):
            g0 = m * 2
            g1 = g0 + 1
            c0 = fire(g0, 0, sems[0])
            c1 = fire(g1, 1, sems[1])
            drain_extract(g0, 0, c0)
            drain_extract(g1, 1, c1)
            return carry

        lax.fori_loop(0, n_groups // 2, body, 0)
        pltpu.sync_copy(
            cols_v, out_hbm.at[:, pl.ds(pl.multiple_of(base, TILE_W), b_per_w)]
        )

    return gather_kernel


def kernel(indices, values):
    idx = indices.astype(jnp.int32)
    n, k = values.shape
    out_t = _make_gather(indices.shape[0], k, n)(values.T, idx)
    return out_t.T


# speedup vs baseline: 1.0728x; 1.0728x over previous
import functools

import jax
import jax.numpy as jnp
from jax import lax
from jax.experimental import pallas as pl
from jax.experimental.pallas import tpu as pltpu
from jax.experimental.pallas import tpu_sc as plsc

LANES = 16
TILE_W = 128


def _make_gather(b: int, k: int, n: int):
    info = plsc.get_sparse_core_info()
    nc, ns = info.num_cores, info.num_subcores
    nw = nc * ns
    b_per_w = b // nw
    n_groups = b_per_w // LANES
    mesh = plsc.VectorSubcoreMesh(core_axis_name="c", subcore_axis_name="s")

    @functools.partial(
        pl.kernel,
        mesh=mesh,
        out_type=jax.ShapeDtypeStruct((k, b), jnp.float32),
        scratch_types=[
            pltpu.VMEM((b_per_w,), jnp.int32),
            pltpu.VMEM((LANES, k, TILE_W), jnp.float32),
            pltpu.VMEM((k, b_per_w), jnp.float32),
            pltpu.SemaphoreType.DMA,
        ],
        compiler_params=pltpu.CompilerParams(
            disable_bounds_checks=True, needs_layout_passes=False
        ),
    )
    def gather_kernel(table_hbm, idx_hbm, out_hbm, idx_v, blk_v, cols_v, sem):
        wid = lax.axis_index("s") * nc + lax.axis_index("c")
        base = wid * b_per_w
        pltpu.sync_copy(idx_hbm.at[pl.ds(base, b_per_w)], idx_v)
        lane_iota = lax.iota(jnp.int32, LANES)

        def body(g, carry):
            vec = idx_v[pl.ds(g * LANES, LANES)]
            copies = []
            for l in range(LANES):
                idx_l = lax.squeeze(lax.slice(vec, (l,), (l + 1,)), (0,))
                toff = pl.multiple_of(
                    (idx_l // TILE_W) * TILE_W, TILE_W
                )
                copies.append(
                    pltpu.async_copy(
                        table_hbm.at[:, pl.ds(toff, TILE_W)],
                        blk_v.at[l],
                        sem,
                    )
                )
            for c in copies:
                c.wait()
            cvec = lax.rem(vec, TILE_W)
            obase = g * LANES
            for j in range(k):
                jvec = jnp.full((LANES,), j, jnp.int32)
                vals = plsc.load_gather(blk_v, [lane_iota, jvec, cvec])
                plsc.store_scatter(
                    cols_v, [jvec, lane_iota + obase], vals
                )
            return carry

        lax.fori_loop(0, n_groups, body, 0)
        pltpu.sync_copy(
            cols_v, out_hbm.at[:, pl.ds(pl.multiple_of(base, TILE_W), b_per_w)]
        )

    return gather_kernel


def kernel(indices, values):
    idx = indices.astype(jnp.int32)
    n, k = values.shape
    out_t = _make_gather(indices.shape[0], k, n)(values.T, idx)
    return out_t.T
